# baseline (speedup 1.0000x reference)
import functools

import jax
import jax.numpy as jnp
from jax import lax
from jax.experimental import pallas as pl
from jax.experimental.pallas import tpu as pltpu
from jax.experimental.pallas import tpu_sc as plsc

VOCAB = 1000000
EMBED_DIM = 4
BATCH = 16384
PACK = 16 // EMBED_DIM
VLINES = VOCAB // PACK

_NUM_CORES = 2
_NUM_SUBCORES = 16
_NUM_WORKERS = _NUM_CORES * _NUM_SUBCORES
_B_PER_W = BATCH // _NUM_WORKERS
_CHUNK = 128
_N_CHUNKS = _B_PER_W // _CHUNK
_LANES = 16

_mesh = plsc.VectorSubcoreMesh(core_axis_name="c", subcore_axis_name="s")


@functools.partial(
    pl.kernel,
    mesh=_mesh,
    compiler_params=pltpu.CompilerParams(
        use_tc_tiling_on_sc=False, needs_layout_passes=False
    ),
    out_type=jax.ShapeDtypeStruct((BATCH * EMBED_DIM,), jnp.float32),
    scratch_types=[
        pltpu.VMEM((_B_PER_W,), jnp.int32),
        pltpu.VMEM((_N_CHUNKS, _CHUNK), jnp.int32),
        pltpu.VMEM((_B_PER_W, 16), jnp.float32),
        pltpu.VMEM((_B_PER_W * EMBED_DIM,), jnp.float32),
        pltpu.SemaphoreType.DMA,
    ],
)
def _embed_gather(e_hbm, table_hbm, out_hbm, idx_v, pidx_v, lines_v, outb_v, sem):
    wid = lax.axis_index("s") * _NUM_CORES + lax.axis_index("c")
    base = wid * _B_PER_W
    pltpu.sync_copy(e_hbm.at[pl.ds(base, _B_PER_W)], idx_v)

    for i in range(_B_PER_W // _LANES):
        v = idx_v[pl.ds(i * _LANES, _LANES)] >> 2
        pidx_v[i * _LANES // _CHUNK, pl.ds((i * _LANES) % _CHUNK, _LANES)] = v

    copies = [
        pltpu.async_copy(
            table_hbm.at[pidx_v.at[j]],
            lines_v.at[pl.ds(j * _CHUNK, _CHUNK)],
            sem,
        )
        for j in range(_N_CHUNKS)
    ]
    for c in copies:
        c.wait()


    lane = lax.iota(jnp.int32, _LANES)
    for i in range(_B_PER_W * EMBED_DIM // _LANES):
        k = (lane >> 2) + i * (_LANES // EMBED_DIM)
        j = lane & 3
        rk = plsc.load_gather(idx_v, [k]) & 3
        vals = plsc.load_gather(lines_v, [k, (rk << 2) + j])
        outb_v[pl.ds(i * _LANES, _LANES)] = vals

    pltpu.sync_copy(
        outb_v, out_hbm.at[pl.ds(base * EMBED_DIM, _B_PER_W * EMBED_DIM)]
    )


def kernel(e, table):
    table_lines = jnp.reshape(table, (VLINES, 16))
    flat = _embed_gather(e.astype(jnp.int32), table_lines)
    return jnp.reshape(flat, (BATCH, EMBED_DIM))
